# baseline (speedup 1.0000x reference)
import jax
import jax.numpy as jnp
from jax import lax
from jax.experimental import pallas as pl
from jax.experimental.pallas import tpu as pltpu
from jax.experimental.pallas import tpu_sc as plsc

NC = 2
NS = 16
NW = NC * NS

BLK = 128
SBLK = 4
NBUF = 2
ROWS_PER_TILE = 244


def _gather_grid(n: int, vocab: int, dim: int):
    mesh = plsc.VectorSubcoreMesh(core_axis_name="c", subcore_axis_name="s")
    satoms = SBLK * BLK
    ns = ROWS_PER_TILE // SBLK
    nrows = -(-n // BLK)
    span = ROWS_PER_TILE * BLK
    rem_rows = nrows - NW * ROWS_PER_TILE
    rem_atoms = n - NW * span
    fetch = ROWS_PER_TILE + rem_rows
    assert ROWS_PER_TILE % SBLK == 0 and rem_rows * BLK >= rem_atoms

    def body(table_hbm, idx_hbm, out_hbm, table_sh, idx_span, rows_v,
             sem_idx, sem_gat, sem_out):
        sid = lax.axis_index("s")
        wid = sid * NC + lax.axis_index("c")
        row0 = wid * ROWS_PER_TILE
        base = wid * span

        idx_fetch = pltpu.make_async_copy(
            idx_hbm.at[pl.ds(row0, fetch)], idx_span, sem_idx)
        idx_fetch.start()

        @pl.when(sid == 0)
        def _():
            pltpu.sync_copy(table_hbm, table_sh)

        idx_fetch.wait()
        plsc.subcore_barrier()

        def gat_copy(s, j):
            return pltpu.make_async_copy(
                table_sh.at[idx_span.at[s * SBLK + j]],
                rows_v.at[s % NBUF, pl.ds(j * BLK, BLK)], sem_gat)

        def out_copy(s):
            return pltpu.make_async_copy(
                rows_v.at[s % NBUF], out_hbm.at[pl.ds(base + s * satoms,
                                                      satoms)], sem_out)

        def slot(s, retire, reclaim):
            if retire:
                for j in range(SBLK):
                    gat_copy(s - 1, j).wait()
                out_copy(s - 1).start()
            if reclaim:
                out_copy(s - NBUF).wait()
            for j in range(SBLK):
                gat_copy(s, j).start()

        for s in range(NBUF):
            slot(s, retire=(s >= 1), reclaim=False)

        def steady(s, carry):
            slot(s, retire=True, reclaim=True)
            return carry

        lax.fori_loop(NBUF, ns, steady, 0)

        for j in range(SBLK):
            gat_copy(ns - 1, j).wait()
        out_copy(ns - 1).start()
        for s in range(ns - NBUF, ns):
            out_copy(s).wait()

        @pl.when(wid == NW - 1)
        def _():
            for j in range(rem_rows):
                take = min(BLK, rem_atoms - j * BLK)
                if take <= 0:
                    break
                tail = pltpu.make_async_copy(
                    table_sh.at[idx_span.at[ROWS_PER_TILE + j]],
                    rows_v.at[0, pl.ds(0, BLK)], sem_gat)
                tail.start()
                tail.wait()
                pltpu.sync_copy(
                    rows_v.at[0, pl.ds(0, take)],
                    out_hbm.at[pl.ds(NW * span + j * BLK, take)])

    return pl.kernel(
        body,
        out_type=jax.ShapeDtypeStruct((n, dim), jnp.float32),
        mesh=mesh,
        scratch_types=[
            pltpu.VMEM_SHARED((vocab, dim), jnp.float32),
            pltpu.VMEM((fetch, BLK), jnp.int32),
            pltpu.VMEM((NBUF, satoms, dim), jnp.float32),
            pltpu.SemaphoreType.DMA,
            pltpu.SemaphoreType.DMA,
            pltpu.SemaphoreType.DMA,
        ],
        compiler_params=pltpu.CompilerParams(use_tc_tiling_on_sc=False),
    )


@jax.jit
def kernel(embeddings, indices):
    n = indices.shape[0]
    dim = embeddings.shape[1]
    nrows = -(-n // BLK)
    idx2d = jnp.pad(indices, (0, nrows * BLK - n)).reshape(nrows, BLK)
    return _gather_grid(n, embeddings.shape[0], dim)(embeddings, idx2d)
